# Initial kernel scaffold; baseline (speedup 1.0000x reference)
import jax
import jax.numpy as jnp
from jax.experimental import pallas as pl


def kernel(token_indices, position_embedding_matrix):
    raise NotImplementedError("write your pallas kernel here")


# speedup vs baseline: 4.9240x; 4.9240x over previous
import functools

import jax
import jax.numpy as jnp
from jax import lax
from jax.experimental import pallas as pl
from jax.experimental.pallas import tpu as pltpu
from jax.experimental.pallas import tpu_sc as plsc

NC = 2
NS = 16
NW = NC * NS

CH = 128
K = 8
ROWS = 200
OUTER = ROWS // K

B = 4096 * 200
BPW = B // NW
D = 64


def _body(idx_hbm, table_hbm, out_hbm, idx_v, rows_v, sem):
    wid = lax.axis_index("s") * NC + lax.axis_index("c")
    base = wid * BPW

    pltpu.sync_copy(idx_hbm.at[wid], idx_v)

    def outer(t, carry):
        handles = []
        for j in range(K):
            r = t * K + j
            h = pltpu.async_copy(
                table_hbm.at[idx_v.at[r]],
                rows_v.at[pl.ds(j * CH, CH)],
                sem,
            )
            handles.append(h)
        for h in handles:
            h.wait()
        pltpu.sync_copy(rows_v, out_hbm.at[pl.ds(base + t * (K * CH), K * CH)])
        return carry

    lax.fori_loop(0, OUTER, outer, 0)


@functools.partial(jax.jit, static_argnums=())
def kernel(token_indices, position_embedding_matrix):
    idx = token_indices.astype(jnp.int32).reshape(NW, ROWS, CH)
    run = pl.kernel(
        _body,
        out_type=jax.ShapeDtypeStruct((B, D), jnp.float32),
        mesh=plsc.VectorSubcoreMesh(core_axis_name="c", subcore_axis_name="s"),
        scratch_types=[
            pltpu.VMEM((ROWS, CH), jnp.int32),
            pltpu.VMEM((K * CH, D), jnp.float32),
            pltpu.SemaphoreType.DMA,
        ],
        compiler_params=pltpu.CompilerParams(use_tc_tiling_on_sc=False),
    )
    out = run(idx, position_embedding_matrix)
    return out.reshape(4096, 200, D)


# speedup vs baseline: 4.9699x; 1.0093x over previous
import functools

import jax
import jax.numpy as jnp
from jax import lax
from jax.experimental import pallas as pl
from jax.experimental.pallas import tpu as pltpu
from jax.experimental.pallas import tpu_sc as plsc

NC = 2
NS = 16
NW = NC * NS

CH = 128
K = 4
CHUNK = K * CH
ROWS = 200
NCHUNK = ROWS // K

B = 4096 * 200
BPW = B // NW
D = 64


def _body(idx_hbm, table_hbm, out_hbm, idx_v, rows_v, gs0, gs1, ws0, ws1):
    wid = lax.axis_index("s") * NC + lax.axis_index("c")
    base = wid * BPW
    gsem = (gs0, gs1)
    wsem = (ws0, ws1)

    pltpu.sync_copy(idx_hbm.at[wid], idx_v)

    def fire_gathers(t, s):
        for j in range(K):
            pltpu.async_copy(
                table_hbm.at[idx_v.at[t * K + j]],
                rows_v.at[s].at[pl.ds(j * CH, CH)],
                gsem[s],
            )

    def drain_gathers(t, s):
        for j in range(K):
            pltpu.make_async_copy(
                table_hbm.at[idx_v.at[t * K + j]],
                rows_v.at[s].at[pl.ds(j * CH, CH)],
                gsem[s],
            ).wait()

    def fire_write(t, s):
        pltpu.async_copy(
            rows_v.at[s],
            out_hbm.at[pl.ds(base + t * CHUNK, CHUNK)],
            wsem[s],
        )

    def wait_write(t, s):
        pltpu.make_async_copy(
            rows_v.at[s],
            out_hbm.at[pl.ds(base + t * CHUNK, CHUNK)],
            wsem[s],
        ).wait()

    fire_gathers(0, 0)
    fire_gathers(1, 1)
    drain_gathers(0, 0)
    fire_write(0, 0)

    def step(i, carry):
        t = 2 * i
        wait_write(t - 2, 0)
        fire_gathers(t, 0)
        drain_gathers(t - 1, 1)
        fire_write(t - 1, 1)
        wait_write(t - 1, 1)
        fire_gathers(t + 1, 1)
        drain_gathers(t, 0)
        fire_write(t, 0)
        return carry

    lax.fori_loop(1, NCHUNK // 2, step, 0)

    drain_gathers(NCHUNK - 1, 1)
    fire_write(NCHUNK - 1, 1)
    wait_write(NCHUNK - 2, 0)
    wait_write(NCHUNK - 1, 1)


@functools.partial(jax.jit, static_argnums=())
def kernel(token_indices, position_embedding_matrix):
    idx = token_indices.astype(jnp.int32).reshape(NW, ROWS, CH)
    run = pl.kernel(
        _body,
        out_type=jax.ShapeDtypeStruct((B, D), jnp.float32),
        mesh=plsc.VectorSubcoreMesh(core_axis_name="c", subcore_axis_name="s"),
        scratch_types=[
            pltpu.VMEM((ROWS, CH), jnp.int32),
            pltpu.VMEM((2, CHUNK, D), jnp.float32),
            pltpu.SemaphoreType.DMA,
            pltpu.SemaphoreType.DMA,
            pltpu.SemaphoreType.DMA,
            pltpu.SemaphoreType.DMA,
        ],
        compiler_params=pltpu.CompilerParams(use_tc_tiling_on_sc=False),
    )
    out = run(idx, position_embedding_matrix)
    return out.reshape(4096, 200, D)


# speedup vs baseline: 5.6001x; 1.1268x over previous
import functools

import jax
import jax.numpy as jnp
from jax import lax
from jax.experimental import pallas as pl
from jax.experimental.pallas import tpu as pltpu
from jax.experimental.pallas import tpu_sc as plsc

NC = 2
NS = 16
NW = NC * NS

CH = 128
K = 4
CHUNK = K * CH
ROWS = 200
NCHUNK = ROWS // K

B = 4096 * 200
BPW = B // NW
D = 64
V = 8193


def _body(idx_hbm, table_hbm, out_hbm, table_sh, idx_v, rows_v,
          gs0, gs1, ws0, ws1):
    sid = lax.axis_index("s")
    wid = sid * NC + lax.axis_index("c")
    base = wid * BPW
    gsem = (gs0, gs1)
    wsem = (ws0, ws1)

    @pl.when(sid == 0)
    def _():
        pltpu.sync_copy(table_hbm, table_sh)

    pltpu.sync_copy(idx_hbm.at[wid], idx_v)
    plsc.subcore_barrier()

    def fire_gathers(t, s):
        for j in range(K):
            pltpu.async_copy(
                table_sh.at[idx_v.at[t * K + j]],
                rows_v.at[s].at[pl.ds(j * CH, CH)],
                gsem[s],
            )

    def drain_gathers(t, s):
        for j in range(K):
            pltpu.make_async_copy(
                table_sh.at[idx_v.at[t * K + j]],
                rows_v.at[s].at[pl.ds(j * CH, CH)],
                gsem[s],
            ).wait()

    def fire_write(t, s):
        pltpu.async_copy(
            rows_v.at[s],
            out_hbm.at[pl.ds(base + t * CHUNK, CHUNK)],
            wsem[s],
        )

    def wait_write(t, s):
        pltpu.make_async_copy(
            rows_v.at[s],
            out_hbm.at[pl.ds(base + t * CHUNK, CHUNK)],
            wsem[s],
        ).wait()

    fire_gathers(0, 0)
    fire_gathers(1, 1)
    drain_gathers(0, 0)
    fire_write(0, 0)

    def step(i, carry):
        t = 2 * i
        wait_write(t - 2, 0)
        fire_gathers(t, 0)
        drain_gathers(t - 1, 1)
        fire_write(t - 1, 1)
        wait_write(t - 1, 1)
        fire_gathers(t + 1, 1)
        drain_gathers(t, 0)
        fire_write(t, 0)
        return carry

    lax.fori_loop(1, NCHUNK // 2, step, 0)

    drain_gathers(NCHUNK - 1, 1)
    fire_write(NCHUNK - 1, 1)
    wait_write(NCHUNK - 2, 0)
    wait_write(NCHUNK - 1, 1)


@functools.partial(jax.jit, static_argnums=())
def kernel(token_indices, position_embedding_matrix):
    idx = token_indices.astype(jnp.int32).reshape(NW, ROWS, CH)
    run = pl.kernel(
        _body,
        out_type=jax.ShapeDtypeStruct((B, D), jnp.float32),
        mesh=plsc.VectorSubcoreMesh(core_axis_name="c", subcore_axis_name="s"),
        scratch_types=[
            pltpu.VMEM_SHARED((V, D), jnp.float32),
            pltpu.VMEM((ROWS, CH), jnp.int32),
            pltpu.VMEM((2, CHUNK, D), jnp.float32),
            pltpu.SemaphoreType.DMA,
            pltpu.SemaphoreType.DMA,
            pltpu.SemaphoreType.DMA,
            pltpu.SemaphoreType.DMA,
        ],
        compiler_params=pltpu.CompilerParams(use_tc_tiling_on_sc=False),
    )
    out = run(idx, position_embedding_matrix)
    return out.reshape(4096, 200, D)
